# baseline (speedup 1.0000x reference)
import numpy as np
import jax
import jax.numpy as jnp
from jax.experimental import pallas as pl
from jax.experimental.pallas import tpu as pltpu

_JNT = np.array([0, 5, 1, 9, 13, 17, 6, 2, 10, 14, 18, 7, 3, 11, 15, 19, 8, 4, 12, 16, 20])
_PERM = (_JNT[:, None] + np.arange(3)[None, :]).flatten()

_ROWS = 16384
_COLS = 63
_KSRC = 24
_BR = 8
_GRID = -(-_COLS // _BR)

_PSEL = np.zeros((_COLS, _KSRC), np.float32)
_PSEL[np.arange(_COLS), _PERM] = 1.0


def _body(p_ref, x_ref, o_ref, z_ref):
    o_ref[...] = jnp.dot(p_ref[...], x_ref[...], preferred_element_type=jnp.float32)
    z_ref[...] = jnp.zeros_like(z_ref)


def kernel(inputs):
    x_t = inputs.T
    x_t = pltpu.with_memory_space_constraint(x_t, pltpu.HBM)
    out_t, z_t = pl.pallas_call(
        _body,
        grid=(_GRID,),
        in_specs=[
            pl.BlockSpec((_BR, _KSRC), lambda i: (i, 0)),
            pl.BlockSpec((_KSRC, _ROWS), lambda i: (0, 0)),
        ],
        out_specs=[
            pl.BlockSpec((_BR, _ROWS), lambda i: (i, 0)),
            pl.BlockSpec((1, _ROWS), lambda i: (0, 0)),
        ],
        out_shape=[
            jax.ShapeDtypeStruct((_COLS, _ROWS), jnp.float32),
            jax.ShapeDtypeStruct((1, _ROWS), jnp.float32),
        ],
        compiler_params=pltpu.CompilerParams(
            dimension_semantics=("arbitrary",),
        ),
    )(jnp.asarray(_PSEL), x_t)
    return (out_t.T, z_t.T)


# speedup vs baseline: 1.2210x; 1.2210x over previous
import numpy as np
import jax
import jax.numpy as jnp
from jax.experimental import pallas as pl
from jax.experimental.pallas import tpu as pltpu

_JNT = np.array([0, 5, 1, 9, 13, 17, 6, 2, 10, 14, 18, 7, 3, 11, 15, 19, 8, 4, 12, 16, 20])
_PERM = (_JNT[:, None] + np.arange(3)[None, :]).flatten()

_ROWS = 16384
_COLS = 63
_KSRC = 24
_CH = 16
_NCH = 4

_OROWS = 64

_PSEL = np.zeros((_OROWS, _KSRC), np.float32)
_PSEL[np.arange(_COLS), _PERM] = 1.0


def _body(p_ref, x_hbm, o_hbm, z_hbm, x_v, o_v0, o_v1, z_v, in_sem, z_sem, out_sems):
    in_cp = pltpu.make_async_copy(x_hbm.at[pl.ds(0, _KSRC), :], x_v, in_sem)
    in_cp.start()
    z_v[...] = jnp.zeros_like(z_v)
    z_cp = pltpu.make_async_copy(z_v, z_hbm, z_sem)
    z_cp.start()
    in_cp.wait()
    bufs = (o_v0, o_v1)

    def _copy(c):
        return pltpu.make_async_copy(
            bufs[c % 2], o_hbm.at[pl.ds(c * _CH, _CH), :], out_sems.at[c % 2],
        )

    for c in range(_NCH):
        if c >= 2:
            _copy(c - 2).wait()
        bufs[c % 2][...] = jnp.dot(
            p_ref[pl.ds(c * _CH, _CH), :], x_v[...],
            preferred_element_type=jnp.float32,
        )
        _copy(c).start()
    for c in range(max(0, _NCH - 2), _NCH):
        _copy(c).wait()
    z_cp.wait()


def kernel(inputs):
    x_t = inputs.T
    x_t = pltpu.with_memory_space_constraint(x_t, pltpu.HBM)
    out_t, z_t = pl.pallas_call(
        _body,
        in_specs=[
            pl.BlockSpec((_OROWS, _KSRC), lambda: (0, 0)),
            pl.BlockSpec(memory_space=pltpu.HBM),
        ],
        out_specs=[
            pl.BlockSpec(memory_space=pltpu.HBM),
            pl.BlockSpec(memory_space=pltpu.HBM),
        ],
        out_shape=[
            jax.ShapeDtypeStruct((_OROWS, _ROWS), jnp.float32),
            jax.ShapeDtypeStruct((1, _ROWS), jnp.float32),
        ],
        scratch_shapes=[
            pltpu.VMEM((_KSRC, _ROWS), jnp.float32),
            pltpu.VMEM((_CH, _ROWS), jnp.float32),
            pltpu.VMEM((_CH, _ROWS), jnp.float32),
            pltpu.VMEM((1, _ROWS), jnp.float32),
            pltpu.SemaphoreType.DMA,
            pltpu.SemaphoreType.DMA,
            pltpu.SemaphoreType.DMA((2,)),
        ],
    )(jnp.asarray(_PSEL), x_t)
    return (out_t.T[:, :_COLS], z_t.T)


# speedup vs baseline: 1.3546x; 1.1094x over previous
import numpy as np
import jax
import jax.numpy as jnp
from jax.experimental import pallas as pl
from jax.experimental.pallas import tpu as pltpu

_JNT = np.array([0, 5, 1, 9, 13, 17, 6, 2, 10, 14, 18, 7, 3, 11, 15, 19, 8, 4, 12, 16, 20])
_PERM = (_JNT[:, None] + np.arange(3)[None, :]).flatten()

_ROWS = 16384
_COLS = 63
_KSRC = 24
_CH = 16
_NCH = 4

_OROWS = 64

_PSEL = np.zeros((_OROWS, _KSRC), np.float32)
_PSEL[np.arange(_COLS), _PERM] = 1.0


def _body(p_ref, x_hbm, o_hbm, z_hbm, x_v, o_v0, o_v1, o_v2, o_v3, z_v, in_sem, z_sem, out_sems):
    in_cp = pltpu.make_async_copy(x_hbm.at[pl.ds(0, _KSRC), :], x_v, in_sem)
    in_cp.start()
    z_v[...] = jnp.zeros_like(z_v)
    z_cp = pltpu.make_async_copy(z_v, z_hbm, z_sem)
    z_cp.start()
    in_cp.wait()
    bufs = (o_v0, o_v1, o_v2, o_v3)

    def _copy(c):
        return pltpu.make_async_copy(
            bufs[c], o_hbm.at[pl.ds(c * _CH, _CH), :], out_sems.at[c],
        )

    for c in range(_NCH):
        bufs[c][...] = jnp.dot(
            p_ref[pl.ds(c * _CH, _CH), :], x_v[...],
            preferred_element_type=jnp.float32,
        )
        _copy(c).start()
    for c in range(_NCH):
        _copy(c).wait()
    z_cp.wait()


def kernel(inputs):
    x_t = inputs.T
    x_t = pltpu.with_memory_space_constraint(x_t, pltpu.HBM)
    out_t, z_t = pl.pallas_call(
        _body,
        in_specs=[
            pl.BlockSpec((_OROWS, _KSRC), lambda: (0, 0)),
            pl.BlockSpec(memory_space=pltpu.HBM),
        ],
        out_specs=[
            pl.BlockSpec(memory_space=pltpu.HBM),
            pl.BlockSpec(memory_space=pltpu.HBM),
        ],
        out_shape=[
            jax.ShapeDtypeStruct((_OROWS, _ROWS), jnp.float32),
            jax.ShapeDtypeStruct((1, _ROWS), jnp.float32),
        ],
        scratch_shapes=[
            pltpu.VMEM((_KSRC, _ROWS), jnp.float32),
            pltpu.VMEM((_CH, _ROWS), jnp.float32),
            pltpu.VMEM((_CH, _ROWS), jnp.float32),
            pltpu.VMEM((_CH, _ROWS), jnp.float32),
            pltpu.VMEM((_CH, _ROWS), jnp.float32),
            pltpu.VMEM((1, _ROWS), jnp.float32),
            pltpu.SemaphoreType.DMA,
            pltpu.SemaphoreType.DMA,
            pltpu.SemaphoreType.DMA((4,)),
        ],
    )(jnp.asarray(_PSEL), x_t)
    return (out_t.T[:, :_COLS], z_t.T)


# speedup vs baseline: 1.6649x; 1.2291x over previous
import numpy as np
import jax
import jax.numpy as jnp
from jax.experimental import pallas as pl
from jax.experimental.pallas import tpu as pltpu

_JNT = np.array([0, 5, 1, 9, 13, 17, 6, 2, 10, 14, 18, 7, 3, 11, 15, 19, 8, 4, 12, 16, 20])
_PERM = (_JNT[:, None] + np.arange(3)[None, :]).flatten()

_ROWS = 16384
_COLS = 63
_KSRC = 24
_OROWS = 64
_BC = 8192
_GRID = _ROWS // _BC

_PSEL = np.zeros((_OROWS, _KSRC), np.float32)
_PSEL[np.arange(_COLS), _PERM] = 1.0


def _body(p_ref, x_ref, o_ref, z_ref):
    o_ref[...] = jnp.dot(p_ref[...], x_ref[...], preferred_element_type=jnp.float32)
    z_ref[...] = jnp.zeros_like(z_ref)


def kernel(inputs):
    x_t = inputs.T
    x_t = pltpu.with_memory_space_constraint(x_t, pltpu.HBM)
    out_t, z_t = pl.pallas_call(
        _body,
        grid=(_GRID,),
        in_specs=[
            pl.BlockSpec((_OROWS, _KSRC), lambda i: (0, 0)),
            pl.BlockSpec((_KSRC, _BC), lambda i: (0, i)),
        ],
        out_specs=[
            pl.BlockSpec((_OROWS, _BC), lambda i: (0, i)),
            pl.BlockSpec((1, _BC), lambda i: (0, i)),
        ],
        out_shape=[
            jax.ShapeDtypeStruct((_OROWS, _ROWS), jnp.float32),
            jax.ShapeDtypeStruct((1, _ROWS), jnp.float32),
        ],
        compiler_params=pltpu.CompilerParams(
            dimension_semantics=("parallel",),
        ),
    )(jnp.asarray(_PSEL), x_t)
    return (out_t.T[:, :_COLS], z_t.T)
